# baseline (speedup 1.0000x reference)
import functools

import jax
import jax.numpy as jnp
from jax import lax
from jax.experimental import pallas as pl
from jax.experimental.pallas import tpu as pltpu
from jax.experimental.pallas import tpu_sc as plsc

NC = 2
NS = 16
NW = NC * NS

CHUNK = 512


def _sc_gather(table, ctx_idx, q_idx):
    V, D = table.shape
    n_ctx = ctx_idx.shape[0]
    n_q = q_idx.shape[0]
    ctx_per_w = n_ctx // NW
    q_per_w = n_q // NW

    mesh = plsc.VectorSubcoreMesh(core_axis_name="c", subcore_axis_name="s")

    @functools.partial(
        pl.kernel,
        mesh=mesh,
        compiler_params=pltpu.CompilerParams(use_tc_tiling_on_sc=False),
        out_type=(
            jax.ShapeDtypeStruct((n_ctx, D), jnp.float32),
            jax.ShapeDtypeStruct((n_q, D), jnp.float32),
        ),
        scratch_types=[
            pltpu.VMEM((CHUNK,), jnp.int32),
            pltpu.VMEM((CHUNK, D), jnp.float32),
            pltpu.SemaphoreType.DMA,
        ],
    )
    def k(table_hbm, ctx_idx_hbm, q_idx_hbm, ctx_out, q_out, idx_v, rows_v, sem):
        wid = lax.axis_index("s") * NC + lax.axis_index("c")

        def gather_slice(idx_hbm, out_hbm, per_w):
            base = wid * per_w

            @pl.loop(0, per_w, step=CHUNK)
            def _(off):
                pltpu.sync_copy(idx_hbm.at[pl.ds(base + off, CHUNK)], idx_v)
                pltpu.async_copy(table_hbm.at[idx_v], rows_v, sem).wait()
                pltpu.sync_copy(rows_v, out_hbm.at[pl.ds(base + off, CHUNK)])

        gather_slice(ctx_idx_hbm, ctx_out, ctx_per_w)
        gather_slice(q_idx_hbm, q_out, q_per_w)

    return k(table, ctx_idx, q_idx)


def kernel(word_embeddings, input_context, input_question):
    B, CL = input_context.shape
    _, QL = input_question.shape
    D = word_embeddings.shape[1]
    ctx_idx = input_context.reshape(-1).astype(jnp.int32)
    q_idx = input_question.reshape(-1).astype(jnp.int32)
    ctx_rows, q_rows = _sc_gather(word_embeddings, ctx_idx, q_idx)
    return (ctx_rows.reshape(B, CL, D), q_rows.reshape(B, QL, D))


# speedup vs baseline: 1.0451x; 1.0451x over previous
import functools

import jax
import jax.numpy as jnp
from jax import lax
from jax.experimental import pallas as pl
from jax.experimental.pallas import tpu as pltpu
from jax.experimental.pallas import tpu_sc as plsc

NC = 2
NS = 16
NW = NC * NS

CHUNK = 640


def _sc_gather(table, ctx_idx, q_idx):
    V, D = table.shape
    n_ctx = ctx_idx.shape[0]
    n_q = q_idx.shape[0]
    ctx_per_w = n_ctx // NW
    q_per_w = n_q // NW

    mesh = plsc.VectorSubcoreMesh(core_axis_name="c", subcore_axis_name="s")

    @functools.partial(
        pl.kernel,
        mesh=mesh,
        compiler_params=pltpu.CompilerParams(use_tc_tiling_on_sc=False),
        out_type=(
            jax.ShapeDtypeStruct((n_ctx, D), jnp.float32),
            jax.ShapeDtypeStruct((n_q, D), jnp.float32),
        ),
        scratch_types=[
            pltpu.VMEM((ctx_per_w,), jnp.int32),
            pltpu.VMEM((CHUNK, D), jnp.float32),
            pltpu.VMEM((CHUNK, D), jnp.float32),
            pltpu.SemaphoreType.DMA,
            pltpu.SemaphoreType.DMA,
            pltpu.SemaphoreType.DMA,
            pltpu.SemaphoreType.DMA,
        ],
    )
    def k(table_hbm, ctx_idx_hbm, q_idx_hbm, ctx_out, q_out,
          idx_v, rows0, rows1, sg0, sg1, sw0, sw1):
        wid = lax.axis_index("s") * NC + lax.axis_index("c")
        bufs = ((rows0, sg0, sw0), (rows1, sg1, sw1))

        def pipe(idx_hbm, out_hbm, per_w):
            base = wid * per_w
            n = per_w // CHUNK
            pltpu.sync_copy(idx_hbm.at[pl.ds(base, per_w)],
                            idx_v.at[pl.ds(0, per_w)])

            def start_gather(g, rows, sg):
                pltpu.async_copy(
                    table_hbm.at[idx_v.at[pl.ds(g * CHUNK, CHUNK)]], rows, sg)

            def wait_gather(rows, sg):
                pltpu.make_async_copy(
                    table_hbm.at[idx_v.at[pl.ds(0, CHUNK)]], rows, sg).wait()

            def start_write(g, rows, sw):
                pltpu.async_copy(
                    rows, out_hbm.at[pl.ds(base + g * CHUNK, CHUNK)], sw)

            def wait_write(rows, sw):
                pltpu.make_async_copy(rows, out_hbm.at[pl.ds(base, CHUNK)],
                                      sw).wait()

            start_gather(0, rows0, sg0)
            start_gather(1, rows1, sg1)

            @pl.loop(0, n, step=2)
            def _(g):
                for j, (rows, sg, sw) in enumerate(bufs):
                    gg = g + j
                    wait_gather(rows, sg)
                    start_write(gg, rows, sw)

                    @pl.when(gg + 2 < n)
                    def _():
                        wait_write(rows, sw)
                        start_gather(gg + 2, rows, sg)

            wait_write(rows0, sw0)
            wait_write(rows1, sw1)

        pipe(ctx_idx_hbm, ctx_out, ctx_per_w)
        pipe(q_idx_hbm, q_out, q_per_w)

    return k(table, ctx_idx, q_idx)


def kernel(word_embeddings, input_context, input_question):
    B, CL = input_context.shape
    _, QL = input_question.shape
    D = word_embeddings.shape[1]
    ctx_idx = input_context.reshape(-1).astype(jnp.int32)
    q_idx = input_question.reshape(-1).astype(jnp.int32)
    ctx_rows, q_rows = _sc_gather(word_embeddings, ctx_idx, q_idx)
    return (ctx_rows.reshape(B, CL, D), q_rows.reshape(B, QL, D))


# speedup vs baseline: 1.3160x; 1.2593x over previous
import functools

import jax
import jax.numpy as jnp
from jax import lax
from jax.experimental import pallas as pl
from jax.experimental.pallas import tpu as pltpu
from jax.experimental.pallas import tpu_sc as plsc

NC = 2
NS = 16
NW = NC * NS


def _sc_gather(table, ctx_idx, q_idx, B, CL, QL):
    V, D = table.shape
    b_per_w = B // NW
    ctx_per_w = b_per_w * CL
    q_per_w = b_per_w * QL

    mesh = plsc.VectorSubcoreMesh(core_axis_name="c", subcore_axis_name="s")

    @functools.partial(
        pl.kernel,
        mesh=mesh,
        compiler_params=pltpu.CompilerParams(use_tc_tiling_on_sc=False),
        out_type=(
            jax.ShapeDtypeStruct((B, CL, 2 * D), jnp.float32),
            jax.ShapeDtypeStruct((B, QL, 2 * D), jnp.float32),
        ),
        scratch_types=[
            pltpu.VMEM((ctx_per_w,), jnp.int32),
            pltpu.VMEM((2 * CL, D), jnp.float32),
            pltpu.VMEM((2 * CL, D), jnp.float32),
            pltpu.SemaphoreType.DMA,
            pltpu.SemaphoreType.DMA,
            pltpu.SemaphoreType.DMA,
            pltpu.SemaphoreType.DMA,
        ],
    )
    def k(table_hbm, ctx_idx_hbm, q_idx_hbm, ctx_out, q_out,
          idx_v, rows0, rows1, sg0, sg1, sw0, sw1):
        wid = lax.axis_index("s") * NC + lax.axis_index("c")
        b_base = wid * b_per_w

        def pipe(idx_hbm, out_hbm, per_w, L, rows_per_chunk):
            C = L * rows_per_chunk
            n = b_per_w // rows_per_chunk
            base = wid * per_w
            pltpu.sync_copy(idx_hbm.at[pl.ds(base, per_w)],
                            idx_v.at[pl.ds(0, per_w)])
            bufs = ((rows0, sg0, sw0), (rows1, sg1, sw1))

            def start_gather(g, rows, sg):
                pltpu.async_copy(
                    table_hbm.at[idx_v.at[pl.ds(g * C, C)]],
                    rows.at[pl.ds(0, C)], sg)

            def wait_gather(rows, sg):
                pltpu.make_async_copy(
                    table_hbm.at[idx_v.at[pl.ds(0, C)]],
                    rows.at[pl.ds(0, C)], sg).wait()

            def start_write(g, rows, sw):
                for r in range(rows_per_chunk):
                    pltpu.async_copy(
                        rows.at[pl.ds(r * L, L)],
                        out_hbm.at[b_base + g * rows_per_chunk + r]
                               .at[:, pl.ds(0, D)], sw)

            def wait_write(rows, sw):
                for r in range(rows_per_chunk):
                    pltpu.make_async_copy(
                        rows.at[pl.ds(r * L, L)],
                        out_hbm.at[b_base].at[:, pl.ds(0, D)], sw).wait()

            start_gather(0, rows0, sg0)
            start_gather(1, rows1, sg1)

            @pl.loop(0, n, step=2)
            def _(g):
                for j, (rows, sg, sw) in enumerate(bufs):
                    gg = g + j
                    wait_gather(rows, sg)
                    start_write(gg, rows, sw)

                    @pl.when(gg + 2 < n)
                    def _():
                        wait_write(rows, sw)
                        start_gather(gg + 2, rows, sg)

            wait_write(rows0, sw0)
            wait_write(rows1, sw1)

        pipe(ctx_idx_hbm, ctx_out, ctx_per_w, CL, 1)
        pipe(q_idx_hbm, q_out, q_per_w, QL, 2)

    return k(table, ctx_idx, q_idx)


def kernel(word_embeddings, input_context, input_question):
    B, CL = input_context.shape
    _, QL = input_question.shape
    ctx_idx = input_context.reshape(-1).astype(jnp.int32)
    q_idx = input_question.reshape(-1).astype(jnp.int32)
    ctx_pad, q_pad = _sc_gather(word_embeddings, ctx_idx, q_idx, B, CL, QL)
    D = word_embeddings.shape[1]
    return (ctx_pad[:, :, :D], q_pad[:, :, :D])
